# baseline (speedup 1.0000x reference)
import functools

import jax
import jax.numpy as jnp
from jax import lax
from jax.experimental import pallas as pl
from jax.experimental.pallas import tpu as pltpu
from jax.experimental.pallas import tpu_sc as plsc

_TOP_K = 128
_MIN_CONF = 0.1
_KEEP_ROWS = 256
_CBLK = 16


def _rowmax_kernel(x_ref, o_ref):
    o_ref[...] = jnp.max(x_ref[...], axis=-1)


def _row_maxima(points_heatmap, nheat):
    B, C, H, W = points_heatmap.shape
    grid = (B, nheat // _CBLK)
    return pl.pallas_call(
        _rowmax_kernel,
        grid=grid,
        in_specs=[pl.BlockSpec((1, _CBLK, H, W), lambda b, i: (b, i, 0, 0))],
        out_specs=pl.BlockSpec((1, _CBLK, H), lambda b, i: (b, i, 0)),
        out_shape=jax.ShapeDtypeStruct((B, nheat, H), points_heatmap.dtype),
    )(points_heatmap)


def _sc_row_gather(table, idx, W):
    info = plsc.get_sparse_core_info()
    nw = info.num_cores * info.num_subcores
    n = idx.shape[0]
    n_per_w = n // nw
    mesh = plsc.VectorSubcoreMesh(core_axis_name="c", subcore_axis_name="s")

    @functools.partial(
        pl.kernel,
        mesh=mesh,
        out_type=jax.ShapeDtypeStruct((n, W), table.dtype),
        scratch_types=[
            pltpu.VMEM((n_per_w,), jnp.int32),
            pltpu.VMEM((n_per_w, W), table.dtype),
            pltpu.SemaphoreType.DMA,
        ],
    )
    def gather_kernel(table_hbm, idx_hbm, out_hbm, idx_v, rows_v, sem):
        wid = lax.axis_index("s") * info.num_cores + lax.axis_index("c")
        base = wid * n_per_w
        pltpu.sync_copy(idx_hbm.at[pl.ds(base, n_per_w)], idx_v)
        pltpu.async_copy(table_hbm.at[idx_v], rows_v, sem).wait()
        pltpu.sync_copy(rows_v, out_hbm.at[pl.ds(base, n_per_w)])

    return gather_kernel(table, idx)


def kernel(points_heatmap):
    B, C, H, W = points_heatmap.shape
    nheat = C - 4

    rowmax = _row_maxima(points_heatmap, nheat).reshape(B, nheat * H)

    _, rid = jax.lax.top_k(rowmax, _KEEP_ROWS)
    rid = jnp.sort(rid, axis=1)
    heat_rows = points_heatmap[:, :nheat].reshape(B * nheat * H, W)
    bidx = jnp.arange(B)[:, None]
    gidx = (rid + bidx * (nheat * H)).reshape(B * _KEEP_ROWS).astype(jnp.int32)
    gathered = _sc_row_gather(heat_rows, gidx, W).reshape(B, _KEEP_ROWS * W)
    scores, gpos = jax.lax.top_k(gathered, _TOP_K)

    flat = rid[bidx, gpos // W] * W + (gpos % W)
    clses = (flat // (H * W)).astype(jnp.int32)
    rem = flat % (H * W)
    ys = (rem // W).astype(jnp.int32)
    xs = (rem % W).astype(jnp.int32)

    reg = points_heatmap[:, C - 4 :, :, :].reshape(B, 4, H * W)
    rvals = jnp.take_along_axis(reg, rem[:, None, :], axis=2)
    off_y, off_x, sz_h, sz_w = rvals[:, 0], rvals[:, 1], rvals[:, 2], rvals[:, 3]

    mask = scores > _MIN_CONF
    obj = jnp.stack(
        [
            ys.astype(jnp.float32) + off_y,
            xs.astype(jnp.float32) + off_x,
            sz_h,
            sz_w,
            clses.astype(jnp.float32),
            scores * mask.astype(jnp.float32),
        ],
        axis=-1,
    )
    return jnp.where(mask[..., None], obj, jnp.zeros_like(obj))
